# baseline (speedup 1.0000x reference)
import functools

import jax
import jax.numpy as jnp
from jax.experimental import pallas as pl
from jax.experimental.pallas import tpu as pltpu

PRE_LEN = 128
B_ROWS = 1024
D = 49152
BR = 64
GRID = B_ROWS // BR
NC = 2
DC = D // NC


def _copy_kernel(idx_ref, t_ref, o_ref):
    i = pl.program_id(1)
    for r in range(BR):
        v = idx_ref[i * BR + r]
        o_ref[r, :] = t_ref[v, :]


@jax.jit
def _run(idx_flat, table):
    grid_spec = pltpu.PrefetchScalarGridSpec(
        num_scalar_prefetch=1,
        grid=(NC, GRID),
        in_specs=[pl.BlockSpec((PRE_LEN, DC), lambda j, i, idx_ref: (0, j))],
        out_specs=pl.BlockSpec((BR, DC), lambda j, i, idx_ref: (i, j)),
    )
    return pl.pallas_call(
        _copy_kernel,
        grid_spec=grid_spec,
        out_shape=jax.ShapeDtypeStruct((B_ROWS, D), jnp.float32),
    )(idx_flat, table)


def kernel(prefix, embedding_weight):
    idx_flat = prefix.reshape(-1).astype(jnp.int32)
    out2 = _run(idx_flat, embedding_weight)
    return out2.reshape(prefix.shape[0], prefix.shape[1], D)


# speedup vs baseline: 1.0452x; 1.0452x over previous
import functools

import jax
import jax.numpy as jnp
from jax.experimental import pallas as pl
from jax.experimental.pallas import tpu as pltpu

PRE_LEN = 128
B_ROWS = 1024
D = 49152
BR = 128
GRID = B_ROWS // BR
NC = 2
DC = D // NC


def _copy_kernel(idx_ref, t_ref, o_ref):
    i = pl.program_id(1)
    for r in range(BR):
        v = idx_ref[i * BR + r]
        o_ref[r, :] = t_ref[v, :]


@jax.jit
def _run(idx_flat, table):
    grid_spec = pltpu.PrefetchScalarGridSpec(
        num_scalar_prefetch=1,
        grid=(NC, GRID),
        in_specs=[pl.BlockSpec((PRE_LEN, DC), lambda j, i, idx_ref: (0, j))],
        out_specs=pl.BlockSpec((BR, DC), lambda j, i, idx_ref: (i, j)),
    )
    return pl.pallas_call(
        _copy_kernel,
        grid_spec=grid_spec,
        out_shape=jax.ShapeDtypeStruct((B_ROWS, D), jnp.float32),
    )(idx_flat, table)


def kernel(prefix, embedding_weight):
    idx_flat = prefix.reshape(-1).astype(jnp.int32)
    out2 = _run(idx_flat, embedding_weight)
    return out2.reshape(prefix.shape[0], prefix.shape[1], D)
